# baseline (speedup 1.0000x reference)
import functools

import numpy as np
import jax
import jax.numpy as jnp
from jax import lax
from jax.experimental import pallas as pl
from jax.experimental.pallas import tpu as pltpu
from jax.experimental.pallas import tpu_sc as plsc

N = 65536
E = 8
H = 128
GEO = 15
APP = 32
LP = 10
LD = 4

BR = 256
NBR = N // BR
BM = 256
NPAD = N + E * BM
NPB = NPAD // BM
NPB_PAD = ((NPB + 15) // 16) * 16
NC, NS = 2, 16
NW = NC * NS
ROWS_W = N // NW
CH = 128
NCH = ROWS_W // CH


def _make_sel(L):
    sel = np.zeros((3, 64), np.float32)
    off = np.zeros((1, 64), np.float32)
    for j in range(3):
        sel[j, j] = 1.0
    for l in range(L):
        for k in range(6):
            c = 3 + 6 * l + k
            sel[k % 3, c] = float(2.0 ** l)
            if k >= 3:
                off[0, c] = np.pi / 2.0
    return sel, off


_SEL1, _OFF1 = _make_sel(LP)
_SEL2, _OFF2 = _make_sel(LD)


def _twopi_split():
    v = np.float64(2.0 * np.pi)
    parts = []
    for _ in range(2):
        f = np.float32(v)
        hi = np.uint32(f.view(np.uint32) & np.uint32(0xFFFFF000)).view(np.float32)
        parts.append(hi)
        v = v - np.float64(hi)
    parts.append(np.float32(v))
    return [float(p) for p in parts]


_P1, _P2, _P3 = _twopi_split()
_INV2PI = float(np.float32(1.0 / (2.0 * np.pi)))


def _sin_accurate(x):
    k = jnp.floor(x * _INV2PI + 0.5)
    r = ((x - k * _P1) - k * _P2) - k * _P3
    r2 = r * r
    p = -1.0 / 1307674368000.0
    p = p * r2 + 1.0 / 6227020800.0
    p = p * r2 - 1.0 / 39916800.0
    p = p * r2 + 1.0 / 362880.0
    p = p * r2 - 1.0 / 5040.0
    p = p * r2 + 1.0 / 120.0
    p = p * r2 - 1.0 / 6.0
    return r + r * r2 * p
_LTRI = np.tril(np.ones((BR, BR), np.float32), k=-1)
_TRIU = np.triu(np.ones((16, 16), np.float32))


def _route_body(pos_ref, dir_ref, app_ref, ct_ref, sel1_ref, off1_ref,
                sel2_ref, off2_ref, ltri_ref, triu_ref,
                enc_ref, a_ref, r_ref, pexcl_ref, bexp_ref, carry_ref):
    i = pl.program_id(0)
    pos = pos_ref[...]
    dirs = dir_ref[...]

    d2 = jnp.zeros((BR, E), jnp.float32)
    for j in range(3):
        t = pos[:, j:j + 1] - ct_ref[j:j + 1, :]
        d2 = d2 + t * t
    mn = jnp.min(d2, axis=1, keepdims=True)
    ee8 = lax.broadcasted_iota(jnp.int32, (BR, E), 1)
    assign = jnp.min(jnp.where(d2 <= mn, ee8, E), axis=1, keepdims=True)

    ee16 = lax.broadcasted_iota(jnp.int32, (BR, 16), 1)
    hmat = (assign == ee16).astype(jnp.float32)

    @pl.when(i == 0)
    def _init():
        carry_ref[...] = jnp.zeros_like(carry_ref)

    c0 = carry_ref[...]
    excl = jnp.dot(ltri_ref[...], hmat, preferred_element_type=jnp.float32)
    r = jnp.sum((excl + c0) * hmat, axis=1, keepdims=True)
    a_ref[...] = assign
    r_ref[...] = r.astype(jnp.int32)
    cnew = c0 + jnp.sum(hmat, axis=0, keepdims=True)
    carry_ref[...] = cnew

    @pl.when(i == NBR - 1)
    def _final():
        padded = jnp.floor((cnew + (BM - 1)) * (1.0 / BM)) * BM
        incl = jnp.dot(padded, triu_ref[...],
                       preferred_element_type=jnp.float32)
        pexcl_ref[...] = (incl - padded).astype(jnp.int32)
        inc_i = incl.astype(jnp.int32)
        kbm = lax.broadcasted_iota(jnp.int32, (1, NPB_PAD), 1) * BM
        acc = jnp.zeros((1, NPB_PAD), jnp.int32)
        for e in range(E - 1):
            acc = acc + (kbm >= inc_i[:, e:e + 1]).astype(jnp.int32)
        bexp_ref[...] = acc

    col = lax.broadcasted_iota(jnp.int32, (BR, 64), 1)
    sinm = ((col - 3) % 6) < 3
    x1 = jnp.dot(pos, sel1_ref[...], preferred_element_type=jnp.float32,
                 precision=lax.Precision.HIGHEST)
    e1 = jnp.where(col < 3, x1,
                   jnp.where(sinm, jnp.sin(x1), jnp.cos(x1)))
    e1 = jnp.where(col == 63, 1.0, e1)
    x2 = jnp.dot(dirs, sel2_ref[...], preferred_element_type=jnp.float32,
                 precision=lax.Precision.HIGHEST)
    e2 = jnp.where(col < 3, x2,
                   jnp.where(sinm, jnp.sin(x2), jnp.cos(x2)))
    e2 = jnp.where(col == 31, 1.0, e2)
    enc_ref[...] = jnp.concatenate([e1, e2[:, :32], app_ref[...]], axis=1)


def _route_stage(positions, directions, appearance_embedding, centroids):
    return pl.pallas_call(
        _route_body,
        grid=(NBR,),
        in_specs=[
            pl.BlockSpec((BR, 3), lambda i: (i, 0)),
            pl.BlockSpec((BR, 3), lambda i: (i, 0)),
            pl.BlockSpec((BR, APP), lambda i: (i, 0)),
            pl.BlockSpec((3, E), lambda i: (0, 0)),
            pl.BlockSpec((3, 64), lambda i: (0, 0)),
            pl.BlockSpec((1, 64), lambda i: (0, 0)),
            pl.BlockSpec((3, 64), lambda i: (0, 0)),
            pl.BlockSpec((1, 64), lambda i: (0, 0)),
            pl.BlockSpec((BR, BR), lambda i: (0, 0)),
            pl.BlockSpec((16, 16), lambda i: (0, 0)),
        ],
        out_specs=[
            pl.BlockSpec((BR, 128), lambda i: (i, 0)),
            pl.BlockSpec((BR, 1), lambda i: (i, 0)),
            pl.BlockSpec((BR, 1), lambda i: (i, 0)),
            pl.BlockSpec((1, 16), lambda i: (0, 0)),
            pl.BlockSpec((1, NPB_PAD), lambda i: (0, 0)),
        ],
        out_shape=[
            jax.ShapeDtypeStruct((N, 128), jnp.float32),
            jax.ShapeDtypeStruct((N, 1), jnp.int32),
            jax.ShapeDtypeStruct((N, 1), jnp.int32),
            jax.ShapeDtypeStruct((1, 16), jnp.int32),
            jax.ShapeDtypeStruct((1, NPB_PAD), jnp.int32),
        ],
        scratch_shapes=[pltpu.VMEM((1, 16), jnp.float32)],
    )(positions, directions, appearance_embedding,
      centroids.T, jnp.asarray(_SEL1), jnp.asarray(_OFF1),
      jnp.asarray(_SEL2), jnp.asarray(_OFF2),
      jnp.asarray(_LTRI), jnp.asarray(_TRIU))


def _finalize_body(a_ref, r_ref, pexcl_ref, dst_ref):
    ee16 = lax.broadcasted_iota(jnp.int32, (BR, 16), 1)
    hm = (a_ref[...] == ee16).astype(jnp.int32)
    dst_ref[...] = r_ref[...] + jnp.sum(
        hm * pexcl_ref[...], axis=1, keepdims=True)


def _finalize_stage(a2, r2, pexcl):
    return pl.pallas_call(
        _finalize_body,
        grid=(NBR,),
        in_specs=[
            pl.BlockSpec((BR, 1), lambda i: (i, 0)),
            pl.BlockSpec((BR, 1), lambda i: (i, 0)),
            pl.BlockSpec((1, 16), lambda i: (0, 0)),
        ],
        out_specs=pl.BlockSpec((BR, 1), lambda i: (i, 0)),
        out_shape=jax.ShapeDtypeStruct((N, 1), jnp.int32),
    )(a2, r2, pexcl)


def _mlp_body(be_ref, enc_ref, w1_ref, w2_ref, db2_ref, wc_ref, wg_ref,
              cw_ref, bw_ref, out_ref):
    enc = enc_ref[...]
    e1 = enc[:, :64]
    e2 = enc[:, 64:]
    h = jnp.maximum(
        jnp.dot(e1, w1_ref[0], preferred_element_type=jnp.float32), 0.0)
    out16 = jnp.dot(h, w2_ref[0], preferred_element_type=jnp.float32) + db2_ref[0]
    hc = jnp.maximum(
        jnp.dot(e2, wc_ref[0], preferred_element_type=jnp.float32)
        + jnp.dot(out16, wg_ref[0], preferred_element_type=jnp.float32), 0.0)
    pre = jnp.dot(hc, cw_ref[0], preferred_element_type=jnp.float32) + bw_ref[0]
    colm = lax.broadcasted_iota(jnp.int32, (BM, 128), 1) == 0
    prew = jnp.where(colm, out16[:, :1], pre)
    res = jnp.where(colm,
                    jnp.exp(jnp.clip(prew, -15.0, 15.0)),
                    jax.nn.sigmoid(prew))
    out_ref[...] = res


def _mlp_call(be, sorted_enc, w1p, w2p, db2_3, wc1da, wg16, cw, bw):
    grid_spec = pltpu.PrefetchScalarGridSpec(
        num_scalar_prefetch=1,
        grid=(NPB,),
        in_specs=[
            pl.BlockSpec((BM, 128), lambda i, be: (i, 0)),
            pl.BlockSpec((1, 64, H), lambda i, be: (be[i], 0, 0)),
            pl.BlockSpec((1, H, 16), lambda i, be: (be[i], 0, 0)),
            pl.BlockSpec((1, 1, 16), lambda i, be: (be[i], 0, 0)),
            pl.BlockSpec((1, 64, H), lambda i, be: (be[i], 0, 0)),
            pl.BlockSpec((1, 16, H), lambda i, be: (be[i], 0, 0)),
            pl.BlockSpec((1, H, 128), lambda i, be: (be[i], 0, 0)),
            pl.BlockSpec((1, 1, 128), lambda i, be: (be[i], 0, 0)),
        ],
        out_specs=pl.BlockSpec((BM, 128), lambda i, be: (i, 0)),
    )
    return pl.pallas_call(
        _mlp_body,
        grid_spec=grid_spec,
        out_shape=jax.ShapeDtypeStruct((NPAD, 128), jnp.float32),
    )(be, sorted_enc, w1p, w2p, db2_3, wc1da, wg16, cw, bw)


def _dispatch_body(dst_hbm, enc_hbm, sorted_hbm, d_v, enc_v, sem):
    wid = lax.axis_index("s") * NC + lax.axis_index("c")
    base0 = wid * ROWS_W

    def chunk(ci, carry):
        base = base0 + ci * CH
        pltpu.sync_copy(dst_hbm.at[pl.ds(base, CH)], d_v)
        pltpu.sync_copy(enc_hbm.at[pl.ds(base, CH), :], enc_v)
        pltpu.async_copy(enc_v, sorted_hbm.at[d_v], sem).wait()
        return carry

    lax.fori_loop(0, NCH, chunk, 0)


def _combine_body(dst_hbm, sres_hbm, out_hbm, idx_v, rows_v, sem):
    wid = lax.axis_index("s") * NC + lax.axis_index("c")
    base0 = wid * ROWS_W

    def chunk(ci, carry):
        base = base0 + ci * CH
        pltpu.sync_copy(dst_hbm.at[pl.ds(base, CH)], idx_v)
        pltpu.async_copy(sres_hbm.at[idx_v], rows_v, sem).wait()
        pltpu.sync_copy(rows_v, out_hbm.at[pl.ds(base, CH), :])
        return carry

    lax.fori_loop(0, NCH, chunk, 0)


@functools.lru_cache(maxsize=None)
def _sc_calls():
    mesh = plsc.VectorSubcoreMesh(
        core_axis_name="c", subcore_axis_name="s",
        num_cores=NC, num_subcores=NS)
    dispatch = pl.kernel(
        _dispatch_body,
        out_type=jax.ShapeDtypeStruct((NPAD, 128), jnp.float32),
        mesh=mesh,
        scratch_types=[
            pltpu.VMEM((CH,), jnp.int32),
            pltpu.VMEM((CH, 128), jnp.float32),
            pltpu.SemaphoreType.DMA,
        ],
    )
    combine = pl.kernel(
        _combine_body,
        out_type=jax.ShapeDtypeStruct((N, 128), jnp.float32),
        mesh=mesh,
        scratch_types=[
            pltpu.VMEM((CH,), jnp.int32),
            pltpu.VMEM((CH, 128), jnp.float32),
            pltpu.SemaphoreType.DMA,
        ],
    )
    return dispatch, combine


def kernel(positions, directions, appearance_embedding, centroids,
           dW1, db1, dW2, db2, cW1, cb1, cW2, cb2):
    w1p = jnp.concatenate([dW1, db1[:, None, :]], axis=1)
    w2p = dW2
    db2_3 = db2[:, None, :]
    wc1da = jnp.concatenate([
        cW1[:, :27], jnp.zeros((E, 4, H), jnp.float32),
        cb1[:, None, :], cW1[:, 42:74]], axis=1)
    wg16 = jnp.concatenate([
        jnp.zeros((E, 1, H), jnp.float32), cW1[:, 27:42]], axis=1)
    cw = jnp.concatenate([
        jnp.zeros((E, H, 1), jnp.float32), cW2,
        jnp.zeros((E, H, 124), jnp.float32)], axis=2)
    bw = jnp.concatenate([
        jnp.zeros((E, 1), jnp.float32), cb2,
        jnp.zeros((E, 124), jnp.float32)], axis=1)[:, None, :]

    enc, a2, r2, pexcl, bexp = _route_stage(
        positions, directions, appearance_embedding, centroids)

    dst = _finalize_stage(a2, r2, pexcl).reshape(N)

    _dispatch_call, _combine_call = _sc_calls()
    sorted_enc = _dispatch_call(dst, enc)

    sres = _mlp_call(bexp.reshape(NPB_PAD), sorted_enc,
                     w1p, w2p, db2_3, wc1da, wg16, cw, bw)

    outg = _combine_call(dst, sres)
    return outg[:, :1], outg[:, 1:4]
